# Initial kernel scaffold; baseline (speedup 1.0000x reference)
import jax
import jax.numpy as jnp
from jax.experimental import pallas as pl


def kernel(x, graph_features, params, edge_index, batch):
    raise NotImplementedError("write your pallas kernel here")


# speedup vs baseline: 1.1053x; 1.1053x over previous
import jax
import jax.numpy as jnp
from jax.experimental import pallas as pl

N = 10000
E = 160000
B = 32
HID = 64
HEADS = 8


def _gat_nomax(x, edge_index, W, a_src, a_dst, b, heads, ch):
    loop = jnp.arange(N, dtype=edge_index.dtype)
    src = jnp.concatenate([edge_index[0], loop])
    dst = jnp.concatenate([edge_index[1], loop])
    h = (x @ W).reshape(N, heads, ch)
    es = (h * a_src[None, :, :]).sum(-1)
    ed = (h * a_dst[None, :, :]).sum(-1)
    e = jax.nn.leaky_relu(es[src] + ed[dst], 0.2)
    ex = jnp.exp(e)
    den = jax.ops.segment_sum(ex, dst, num_segments=N)
    raw = jax.ops.segment_sum(h[src] * ex[:, :, None], dst, num_segments=N)
    out = raw / den[:, :, None]
    return out.reshape(N, heads * ch) + b


def _bn(x, g, b):
    return x / jnp.sqrt(1.0 + 1e-5) * g + b


def _mlp_kernel(c_ref, gf_ref, wg_ref, bg_ref, w1_ref, b1_ref, g1_ref, be1_ref,
                w2_ref, b2_ref, g2_ref, be2_ref, w3_ref, b3_ref, o_ref):
    gf = jnp.maximum(gf_ref[...] @ wg_ref[...] + bg_ref[...], 0.0)
    c = jnp.concatenate([c_ref[...], gf], axis=1)
    z = c @ w1_ref[...] + b1_ref[...]
    z = jnp.maximum(_bn(z, g1_ref[...], be1_ref[...]), 0.0)
    z = z @ w2_ref[...] + b2_ref[...]
    z = jnp.maximum(_bn(z, g2_ref[...], be2_ref[...]), 0.0)
    o_ref[...] = z @ w3_ref[...] + b3_ref[...]


def kernel(x, graph_features, params, edge_index, batch):
    p = params
    h = _gat_nomax(x, edge_index, p['W0'], p['as0'], p['ad0'], p['b0'], HEADS, HID)
    h = jax.nn.relu(_bn(h, p['g0'], p['be0']))
    h2 = _gat_nomax(h, edge_index, p['W1'], p['as1'], p['ad1'], p['b1'], HEADS, HID)
    h2 = jax.nn.relu(_bn(h2, p['g1'], p['be1']))
    h = h + h2
    h3 = _gat_nomax(h, edge_index, p['W2'], p['as2'], p['ad2'], p['b2'], 1, HID)
    h = jax.nn.relu(_bn(h3, p['g2'], p['be2']))
    counts = jax.ops.segment_sum(jnp.ones((N, 1), dtype=x.dtype), batch, num_segments=B)
    mean_pool = jax.ops.segment_sum(h, batch, num_segments=B) / jnp.maximum(counts, 1.0)
    max_pool = jax.ops.segment_max(h, batch, num_segments=B)
    c = jnp.concatenate([mean_pool, max_pool], axis=1)
    out = pl.pallas_call(
        _mlp_kernel,
        out_shape=jax.ShapeDtypeStruct((B, 2), jnp.float32),
    )(c, graph_features, p['Wg'], p['bg'], p['Wr1'], p['br1'], p['gr1'], p['ber1'],
      p['Wr2'], p['br2'], p['gr2'], p['ber2'], p['Wr3'], p['br3'])
    return out


# speedup vs baseline: 15.9843x; 14.4609x over previous
import functools

import jax
import jax.numpy as jnp
from jax import lax
from jax.experimental import pallas as pl
from jax.experimental.pallas import tpu as pltpu
from jax.experimental.pallas import tpu_sc as plsc

N = 10000
E = 160000
B = 32
HID = 64
HEADS = 8
D1 = HID * HEADS

NC, NS, LANES = 2, 16, 16
NW = NC * NS
KA = 256
E_TOT = E + N
EP = ((E_TOT + NW * KA - 1) // (NW * KA)) * (NW * KA)
EW = EP // NW
NBLK = EW // KA
NP = 10240
NROWS_T = NP // NS


def _att_body(zeros_hbm, es_hbm, ed_hbm, src_hbm, dst_hbm, ext_hbm, den_hbm,
              sidx, didx, esb, edb, extb, den_sp, sem_s, sem_d):
    cid = lax.axis_index("c")
    sid = lax.axis_index("s")
    wid = sid * NC + cid
    rows0 = sid * NROWS_T
    pltpu.sync_copy(zeros_hbm.at[pl.ds(rows0, NROWS_T)], den_sp.at[pl.ds(rows0, NROWS_T)])
    plsc.subcore_barrier()
    lanes = lax.iota(jnp.int32, 16)

    def blk(i, carry):
        base = wid * EW + i * KA
        pltpu.sync_copy(src_hbm.at[pl.ds(base, KA)], sidx)
        pltpu.sync_copy(dst_hbm.at[pl.ds(base, KA)], didx)
        cs = pltpu.async_copy(es_hbm.at[sidx], esb, sem_s)
        cd = pltpu.async_copy(ed_hbm.at[didx], edb, sem_d)
        cs.wait()
        cd.wait()

        def edge(e, c2):
            v = esb[e, :] + edb[e, :]
            v = jnp.maximum(v, 0.2 * v)
            v = jnp.exp(v)
            v = jnp.where(base + e < E_TOT, v, jnp.zeros_like(v))
            edb[e, :] = v
            plsc.store_scatter(extb, [lanes * KA + e], v, mask=lanes < HEADS)
            return c2

        lax.fori_loop(0, KA, edge, 0, unroll=2)
        pltpu.sync_copy(edb, den_sp.at[didx], add=True)
        for hh in range(HEADS):
            pltpu.sync_copy(extb.at[pl.ds(hh * KA, KA)],
                            ext_hbm.at[hh, pl.ds(base, KA)])
        return carry

    lax.fori_loop(0, NBLK, blk, 0)
    plsc.subcore_barrier()
    pltpu.sync_copy(den_sp.at[pl.ds(rows0, NROWS_T)],
                    den_hbm.at[cid, pl.ds(rows0, NROWS_T)])


_att_call = functools.partial(
    pl.kernel,
    out_type=[jax.ShapeDtypeStruct((HEADS, EP), jnp.float32),
              jax.ShapeDtypeStruct((NC, NP, 16), jnp.float32)],
    mesh=plsc.VectorSubcoreMesh(core_axis_name="c", subcore_axis_name="s"),
    compiler_params=pltpu.CompilerParams(needs_layout_passes=False,
                                         use_tc_tiling_on_sc=False),
    scratch_types=[
        pltpu.VMEM((KA,), jnp.int32),
        pltpu.VMEM((KA,), jnp.int32),
        pltpu.VMEM((KA, 16), jnp.float32),
        pltpu.VMEM((KA, 16), jnp.float32),
        pltpu.VMEM((HEADS * KA,), jnp.float32),
        pltpu.VMEM_SHARED((NP, 16), jnp.float32),
        pltpu.SemaphoreType.DMA,
        pltpu.SemaphoreType.DMA,
    ],
)


KB = 128
EH = EP // NC
ET = EH // NS
NBLKB = ET // KB


def _make_agg(fc, nchunk, heads):
    hpc = fc // HID

    def body(*refs):
        harr = refs[:nchunk]
        ext_hbm, src_hbm, dst_hbm, zeros_hbm = refs[nchunk:nchunk + 4]
        outs = refs[nchunk + 4:2 * nchunk + 4]
        sidx, didx, rowsb, exab, exbb, acc_sp, sem = refs[2 * nchunk + 4:]
        cid = lax.axis_index("c")
        sid = lax.axis_index("s")
        rows0 = sid * NROWS_T
        for ci in range(nchunk):
            pltpu.sync_copy(zeros_hbm.at[pl.ds(rows0, NROWS_T)],
                            acc_sp.at[pl.ds(rows0, NROWS_T)])
            plsc.subcore_barrier()

            def blk(i, carry, ci=ci):
                base = cid * EH + sid * ET + i * KB
                pltpu.sync_copy(src_hbm.at[pl.ds(base, KB)], sidx)
                pltpu.sync_copy(dst_hbm.at[pl.ds(base, KB)], didx)
                cg = pltpu.async_copy(harr[ci].at[sidx], rowsb, sem)
                ha = min(ci * hpc, heads - 1)
                hb = min(ci * hpc + 1, heads - 1)
                pltpu.sync_copy(ext_hbm.at[ha, pl.ds(base, KB)], exab)
                if hpc > 1:
                    pltpu.sync_copy(ext_hbm.at[hb, pl.ds(base, KB)], exbb)
                cg.wait()

                def grp(g, c2):
                    e0 = g * 16
                    av = exab[pl.ds(e0, 16)]
                    bv = exbb[pl.ds(e0, 16)] if hpc > 1 else av
                    for j in range(16):
                        e = e0 + j
                        a = av[j]
                        for q in range(min(4, fc // 16)):
                            rowsb[e, pl.ds(q * 16, 16)] = rowsb[e, pl.ds(q * 16, 16)] * a
                        if hpc > 1:
                            bb = bv[j]
                            for q in range(4, 8):
                                rowsb[e, pl.ds(q * 16, 16)] = rowsb[e, pl.ds(q * 16, 16)] * bb
                    return c2

                lax.fori_loop(0, KB // 16, grp, 0)
                pltpu.sync_copy(rowsb, acc_sp.at[didx], add=True)
                return carry

            lax.fori_loop(0, NBLKB, blk, 0)
            plsc.subcore_barrier()
            pltpu.sync_copy(acc_sp.at[pl.ds(rows0, NROWS_T)],
                            outs[ci].at[cid, pl.ds(rows0, NROWS_T)])

    call = functools.partial(
        pl.kernel,
        out_type=[jax.ShapeDtypeStruct((NC, NP, fc), jnp.float32)
                  for _ in range(nchunk)],
        mesh=plsc.VectorSubcoreMesh(core_axis_name="c", subcore_axis_name="s"),
        compiler_params=pltpu.CompilerParams(needs_layout_passes=False,
                                             use_tc_tiling_on_sc=False),
        scratch_types=[
            pltpu.VMEM((KB,), jnp.int32),
            pltpu.VMEM((KB,), jnp.int32),
            pltpu.VMEM((KB, fc), jnp.float32),
            pltpu.VMEM((KB,), jnp.float32),
            pltpu.VMEM((KB,), jnp.float32),
            pltpu.VMEM_SHARED((NP, fc), jnp.float32),
            pltpu.SemaphoreType.DMA,
        ],
    )
    return call(body)


def _sc_aggregate(h, ext, src_p, dst_p, heads, ch):
    d = heads * ch
    fc = min(d, 128)
    nchunk = d // fc
    agg = _make_agg(fc, nchunk, heads)
    harr = [h[:, ci * fc:(ci + 1) * fc] for ci in range(nchunk)]
    zeros = jnp.zeros((NP, fc), jnp.float32)
    outs = agg(*harr, ext, src_p, dst_p, zeros)
    raw = jnp.concatenate([o[0, :N] + o[1, :N] for o in outs], axis=1)
    return raw


def _sc_attention(es, ed, src_p, dst_p):
    h = es.shape[1]
    esp = jnp.zeros((N, 16), jnp.float32).at[:, :h].set(es)
    edp = jnp.zeros((N, 16), jnp.float32).at[:, :h].set(ed)
    zeros = jnp.zeros((NP, 16), jnp.float32)
    ext, den = _att_call(_att_body)(zeros, esp, edp, src_p, dst_p)
    return ext, den[0, :N, :h] + den[1, :N, :h]


def _gat_sc(x, src_p, dst_p, src, dst, W, a_src, a_dst, b, heads, ch):
    h = (x @ W).reshape(N, heads, ch)
    es = (h * a_src[None, :, :]).sum(-1)
    ed = (h * a_dst[None, :, :]).sum(-1)
    ext, den = _sc_attention(es, ed, src_p, dst_p)
    raw = _sc_aggregate(h.reshape(N, heads * ch), ext, src_p, dst_p, heads, ch)
    out = raw.reshape(N, heads, ch) / den[:, :, None]
    return out.reshape(N, heads * ch) + b


def _bn(x, g, b):
    return x / jnp.sqrt(1.0 + 1e-5) * g + b


def kernel(x, graph_features, params, edge_index, batch):
    p = params
    loop = jnp.arange(N, dtype=edge_index.dtype)
    src = jnp.concatenate([edge_index[0], loop])
    dst = jnp.concatenate([edge_index[1], loop])
    padv = jnp.zeros((EP - E_TOT,), edge_index.dtype)
    src_p = jnp.concatenate([src, padv])
    dst_p = jnp.concatenate([dst, padv])

    h = _gat_sc(x, src_p, dst_p, src, dst, p['W0'], p['as0'], p['ad0'], p['b0'], HEADS, HID)
    h = jax.nn.relu(_bn(h, p['g0'], p['be0']))
    h2 = _gat_sc(h, src_p, dst_p, src, dst, p['W1'], p['as1'], p['ad1'], p['b1'], HEADS, HID)
    h2 = jax.nn.relu(_bn(h2, p['g1'], p['be1']))
    h = h + h2
    h3 = _gat_sc(h, src_p, dst_p, src, dst, p['W2'], p['as2'], p['ad2'], p['b2'], 1, HID)
    h = jax.nn.relu(_bn(h3, p['g2'], p['be2']))
    counts = jax.ops.segment_sum(jnp.ones((N, 1), dtype=x.dtype), batch, num_segments=B)
    mean_pool = jax.ops.segment_sum(h, batch, num_segments=B) / jnp.maximum(counts, 1.0)
    max_pool = jax.ops.segment_max(h, batch, num_segments=B)
    gf = jax.nn.relu(graph_features @ p['Wg'] + p['bg'])
    c = jnp.concatenate([mean_pool, max_pool, gf], axis=1)
    z = jax.nn.relu(_bn(c @ p['Wr1'] + p['br1'], p['gr1'], p['ber1']))
    z = jax.nn.relu(_bn(z @ p['Wr2'] + p['br2'], p['gr2'], p['ber2']))
    return z @ p['Wr3'] + p['br3']


# speedup vs baseline: 17.2696x; 1.0804x over previous
import functools
import math

import jax
import jax.numpy as jnp
from jax import lax
from jax.experimental import pallas as pl
from jax.experimental.pallas import tpu as pltpu
from jax.experimental.pallas import tpu_sc as plsc

N = 10000
E = 160000
B = 32
HID = 64
HEADS = 8
D1 = HID * HEADS

NC, NS, LANES = 2, 16, 16
NW = NC * NS
KA = 256
E_TOT = E + N
EP = ((E_TOT + NW * KA - 1) // (NW * KA)) * (NW * KA)
EW = EP // NW
NBLK = EW // KA
NP = 10240
NROWS_T = NP // NS


def _att_body(zeros_hbm, es_hbm, ed_hbm, src_hbm, dst_hbm, ext_hbm, den_hbm,
              sidx, didx, esb, edb, extb, den_sp, sem_s, sem_d):
    cid = lax.axis_index("c")
    sid = lax.axis_index("s")
    wid = sid * NC + cid
    rows0 = sid * NROWS_T
    pltpu.sync_copy(zeros_hbm.at[pl.ds(rows0, NROWS_T)], den_sp.at[pl.ds(rows0, NROWS_T)])
    plsc.subcore_barrier()
    lanes = lax.iota(jnp.int32, 16)

    def blk(i, carry):
        base = wid * EW + i * KA
        pltpu.sync_copy(src_hbm.at[pl.ds(base, KA)], sidx)
        pltpu.sync_copy(dst_hbm.at[pl.ds(base, KA)], didx)
        cs = pltpu.async_copy(es_hbm.at[sidx], esb, sem_s)
        cd = pltpu.async_copy(ed_hbm.at[didx], edb, sem_d)
        cs.wait()
        cd.wait()

        def edge(e, c2):
            v = esb[e, :] + edb[e, :]
            v = jnp.maximum(v, 0.2 * v)
            v = jnp.exp(v)
            v = jnp.where(base + e < E_TOT, v, jnp.zeros_like(v))
            edb[e, :] = v
            plsc.store_scatter(extb, [lanes * KA + e], v, mask=lanes < HEADS)
            return c2

        lax.fori_loop(0, KA, edge, 0, unroll=2)
        pltpu.sync_copy(edb, den_sp.at[didx], add=True)
        for hh in range(HEADS):
            pltpu.sync_copy(extb.at[pl.ds(hh * KA, KA)],
                            ext_hbm.at[hh, pl.ds(base, KA)])
        return carry

    lax.fori_loop(0, NBLK, blk, 0)
    plsc.subcore_barrier()
    pltpu.sync_copy(den_sp.at[pl.ds(rows0, NROWS_T)],
                    den_hbm.at[cid, pl.ds(rows0, NROWS_T)])


_att_call = functools.partial(
    pl.kernel,
    out_type=[jax.ShapeDtypeStruct((HEADS, EP), jnp.float32),
              jax.ShapeDtypeStruct((NC, NP, 16), jnp.float32)],
    mesh=plsc.VectorSubcoreMesh(core_axis_name="c", subcore_axis_name="s"),
    compiler_params=pltpu.CompilerParams(needs_layout_passes=False,
                                         use_tc_tiling_on_sc=False),
    scratch_types=[
        pltpu.VMEM((KA,), jnp.int32),
        pltpu.VMEM((KA,), jnp.int32),
        pltpu.VMEM((KA, 16), jnp.float32),
        pltpu.VMEM((KA, 16), jnp.float32),
        pltpu.VMEM((HEADS * KA,), jnp.float32),
        pltpu.VMEM_SHARED((NP, 16), jnp.float32),
        pltpu.SemaphoreType.DMA,
        pltpu.SemaphoreType.DMA,
    ],
)


KB = 128
EH = EP // NC
ET = EH // NS
NBLKB = ET // KB


def _make_agg(fc, nchunk, heads):
    hpc = fc // HID

    def body(*refs):
        harr = refs[:nchunk]
        ext_hbm, src_hbm, dst_hbm, zeros_hbm = refs[nchunk:nchunk + 4]
        outs = refs[nchunk + 4:2 * nchunk + 4]
        sidx, didx, rowsb, exab, exbb, acc_sp, sem = refs[2 * nchunk + 4:]
        cid = lax.axis_index("c")
        sid = lax.axis_index("s")
        rows0 = sid * NROWS_T
        for ci in range(nchunk):
            pltpu.sync_copy(zeros_hbm.at[pl.ds(rows0, NROWS_T)],
                            acc_sp.at[pl.ds(rows0, NROWS_T)])
            plsc.subcore_barrier()

            def blk(i, carry, ci=ci):
                base = cid * EH + sid * ET + i * KB
                pltpu.sync_copy(src_hbm.at[pl.ds(base, KB)], sidx)
                pltpu.sync_copy(dst_hbm.at[pl.ds(base, KB)], didx)
                cg = pltpu.async_copy(harr[ci].at[sidx], rowsb, sem)
                ha = min(ci * hpc, heads - 1)
                hb = min(ci * hpc + 1, heads - 1)
                pltpu.sync_copy(ext_hbm.at[ha, pl.ds(base, KB)], exab)
                if hpc > 1:
                    pltpu.sync_copy(ext_hbm.at[hb, pl.ds(base, KB)], exbb)
                cg.wait()

                def grp(g, c2):
                    e0 = g * 16
                    av = exab[pl.ds(e0, 16)]
                    bv = exbb[pl.ds(e0, 16)] if hpc > 1 else av
                    for j in range(16):
                        e = e0 + j
                        a = av[j]
                        for q in range(min(4, fc // 16)):
                            rowsb[e, pl.ds(q * 16, 16)] = rowsb[e, pl.ds(q * 16, 16)] * a
                        if hpc > 1:
                            bb = bv[j]
                            for q in range(4, 8):
                                rowsb[e, pl.ds(q * 16, 16)] = rowsb[e, pl.ds(q * 16, 16)] * bb
                    return c2

                lax.fori_loop(0, KB // 16, grp, 0)
                pltpu.sync_copy(rowsb, acc_sp.at[didx], add=True)
                return carry

            lax.fori_loop(0, NBLKB, blk, 0)
            plsc.subcore_barrier()
            pltpu.sync_copy(acc_sp.at[pl.ds(rows0, NROWS_T)],
                            outs[ci].at[cid, pl.ds(rows0, NROWS_T)])

    call = functools.partial(
        pl.kernel,
        out_type=[jax.ShapeDtypeStruct((NC, NP, fc), jnp.float32)
                  for _ in range(nchunk)],
        mesh=plsc.VectorSubcoreMesh(core_axis_name="c", subcore_axis_name="s"),
        compiler_params=pltpu.CompilerParams(needs_layout_passes=False,
                                             use_tc_tiling_on_sc=False),
        scratch_types=[
            pltpu.VMEM((KB,), jnp.int32),
            pltpu.VMEM((KB,), jnp.int32),
            pltpu.VMEM((KB, fc), jnp.float32),
            pltpu.VMEM((KB,), jnp.float32),
            pltpu.VMEM((KB,), jnp.float32),
            pltpu.VMEM_SHARED((NP, fc), jnp.float32),
            pltpu.SemaphoreType.DMA,
        ],
    )
    return call(body)


def _sc_agg_chunks(harr, ext, src_p, dst_p, heads):
    fc = harr[0].shape[1]
    agg = _make_agg(fc, len(harr), heads)
    zeros = jnp.zeros((NP, fc), jnp.float32)
    outs = agg(*harr, ext, src_p, dst_p, zeros)
    return list(outs) if isinstance(outs, (list, tuple)) else [outs]


def _sc_attention(esp, edp, src_p, dst_p):
    zeros = jnp.zeros((NP, 16), jnp.float32)
    ext, den = _att_call(_att_body)(zeros, esp, edp, src_p, dst_p)
    return ext, den


RB = 1000
GRID = N // RB
PREC = jax.lax.Precision.HIGHEST
BNS = 1.0 / math.sqrt(1.0 + 1e-5)


def _dot(a, b):
    return jax.lax.dot_general(a, b, (((1,), (0,)), ((), ())),
                               precision=PREC, preferred_element_type=jnp.float32)


def _proj_outs(h, as_ref, ad_ref, houts, es_ref, ed_ref):
    nck = len(houts)
    for ci in range(nck):
        houts[ci][...] = h[:, ci * 128:(ci + 1) * 128] if nck > 1 else h
    es_ref[...] = _dot(h, as_ref[...])
    ed_ref[...] = _dot(h, ad_ref[...])


def _t0_body(x_ref, w_ref, as_ref, ad_ref, h0, h1, h2, h3, es_ref, ed_ref):
    h = _dot(x_ref[...], w_ref[...])
    _proj_outs(h, as_ref, ad_ref, [h0, h1, h2, h3], es_ref, ed_ref)


def _assemble(raws, den_ref, bv, gv, bev):
    den = den_ref[0] + den_ref[1]
    zs = []
    for ci in range(4):
        r = raws[ci][0] + raws[ci][1]
        da = den[:, 2 * ci:2 * ci + 1]
        db = den[:, 2 * ci + 1:2 * ci + 2]
        z = jnp.concatenate([r[:, :64] / da, r[:, 64:] / db], axis=1)
        z = z + bv[:, ci * 128:(ci + 1) * 128]
        z = z * (gv[:, ci * 128:(ci + 1) * 128] * BNS) + bev[:, ci * 128:(ci + 1) * 128]
        zs.append(jnp.maximum(z, 0.0))
    return jnp.concatenate(zs, axis=1)


def _t1_body(r0, r1, r2, r3, den_ref, bv, gv, bev, w_ref, as_ref, ad_ref,
             z0, z1, z2, z3, h0, h1, h2, h3, es_ref, ed_ref):
    z = _assemble([r0, r1, r2, r3], den_ref, bv[...], gv[...], bev[...])
    for ci, zr in enumerate([z0, z1, z2, z3]):
        zr[...] = z[:, ci * 128:(ci + 1) * 128]
    h = _dot(z, w_ref[...])
    _proj_outs(h, as_ref, ad_ref, [h0, h1, h2, h3], es_ref, ed_ref)


def _t2_body(r0, r1, r2, r3, den_ref, zr0, zr1, zr2, zr3, bv, gv, bev,
             w_ref, as_ref, ad_ref, h2out, es_ref, ed_ref):
    z = _assemble([r0, r1, r2, r3], den_ref, bv[...], gv[...], bev[...])
    res = jnp.concatenate([zr0[...], zr1[...], zr2[...], zr3[...]], axis=1)
    h = _dot(res + z, w_ref[...])
    _proj_outs(h, as_ref, ad_ref, [h2out], es_ref, ed_ref)


def _t3_body(r_ref, den_ref, bat_ref, gf_ref, b2v, g2v, be2v,
             wg, bgv, wr1, br1, gr1, ber1, wr2, br2, gr2, ber2, wr3, br3,
             out_ref, ssum, smax, scnt):
    pid = pl.program_id(0)

    @pl.when(pid == 0)
    def _():
        ssum[...] = jnp.zeros_like(ssum)
        smax[...] = jnp.zeros_like(smax)
        scnt[...] = jnp.zeros_like(scnt)

    den = den_ref[0, :, :1] + den_ref[1, :, :1]
    h = (r_ref[0] + r_ref[1]) / den + b2v[...]
    h = jnp.maximum(h * (g2v[...] * BNS) + be2v[...], 0.0)
    bat = bat_ref[...]
    mask = (bat == jax.lax.broadcasted_iota(jnp.int32, (RB, B), 1)).astype(jnp.float32)
    ssum[...] += jax.lax.dot_general(mask, h, (((0,), (0,)), ((), ())),
                                     precision=PREC, preferred_element_type=jnp.float32)
    scnt[...] += jnp.sum(mask, axis=0, keepdims=True)
    for b in range(B):
        cand = jnp.max(h * mask[:, b:b + 1], axis=0, keepdims=True)
        smax[pl.ds(b, 1), :] = jnp.maximum(smax[pl.ds(b, 1), :], cand)

    @pl.when(pid == GRID - 1)
    def _():
        cnt = jnp.maximum(scnt[...], 1.0).reshape(B, 1)
        mean = ssum[...] / cnt
        gfv = jnp.maximum(_dot(gf_ref[...], wg[...]) + bgv[...], 0.0)
        c = jnp.concatenate([mean, smax[...], gfv], axis=1)
        zz = _dot(c, wr1[...]) + br1[...]
        zz = jnp.maximum(zz * (gr1[...] * BNS) + ber1[...], 0.0)
        zz = _dot(zz, wr2[...]) + br2[...]
        zz = jnp.maximum(zz * (gr2[...] * BNS) + ber2[...], 0.0)
        out_ref[...] = _dot(zz, wr3[...]) + br3[...]


def _row_spec(w):
    return pl.BlockSpec((RB, w), lambda i: (i, 0))


def _const_spec(shape):
    nd = len(shape)
    return pl.BlockSpec(shape, lambda i, _n=nd: (0,) * _n)


def _nc_spec(w):
    return pl.BlockSpec((NC, RB, w), lambda i: (0, i, 0))


def _tc0(x8, w0p, as0p, ad0p):
    return pl.pallas_call(
        _t0_body,
        grid=(GRID,),
        in_specs=[_row_spec(8), _const_spec((8, D1)), _const_spec((D1, 16)),
                  _const_spec((D1, 16))],
        out_specs=[_row_spec(128)] * 4 + [_row_spec(16)] * 2,
        out_shape=[jax.ShapeDtypeStruct((N, 128), jnp.float32)] * 4
        + [jax.ShapeDtypeStruct((N, 16), jnp.float32)] * 2,
    )(x8, w0p, as0p, ad0p)


def _tc1(raws, den, bv, gv, bev, w, asp, adp):
    return pl.pallas_call(
        _t1_body,
        grid=(GRID,),
        in_specs=[_nc_spec(128)] * 4 + [_nc_spec(16)]
        + [_const_spec((1, D1))] * 3
        + [_const_spec((D1, D1)), _const_spec((D1, 16)), _const_spec((D1, 16))],
        out_specs=[_row_spec(128)] * 8 + [_row_spec(16)] * 2,
        out_shape=[jax.ShapeDtypeStruct((N, 128), jnp.float32)] * 8
        + [jax.ShapeDtypeStruct((N, 16), jnp.float32)] * 2,
    )(*raws, den, bv, gv, bev, w, asp, adp)


def _tc2(raws, den, zres, bv, gv, bev, w, asp, adp):
    return pl.pallas_call(
        _t2_body,
        grid=(GRID,),
        in_specs=[_nc_spec(128)] * 4 + [_nc_spec(16)] + [_row_spec(128)] * 4
        + [_const_spec((1, D1))] * 3
        + [_const_spec((D1, HID)), _const_spec((HID, 16)), _const_spec((HID, 16))],
        out_specs=[_row_spec(64)] + [_row_spec(16)] * 2,
        out_shape=[jax.ShapeDtypeStruct((N, 64), jnp.float32)]
        + [jax.ShapeDtypeStruct((N, 16), jnp.float32)] * 2,
    )(*raws, den, *zres, bv, gv, bev, w, asp, adp)


def _tc3(raw2, den2, bat2, gf, p):
    consts = [jnp.reshape(p['b2'], (1, HID)), jnp.reshape(p['g2'], (1, HID)),
              jnp.reshape(p['be2'], (1, HID)), p['Wg'],
              jnp.reshape(p['bg'], (1, HID // 2)), p['Wr1'],
              jnp.reshape(p['br1'], (1, HID)), jnp.reshape(p['gr1'], (1, HID)),
              jnp.reshape(p['ber1'], (1, HID)), p['Wr2'],
              jnp.reshape(p['br2'], (1, HID // 2)), jnp.reshape(p['gr2'], (1, HID // 2)),
              jnp.reshape(p['ber2'], (1, HID // 2)), p['Wr3'],
              jnp.reshape(p['br3'], (1, 2))]
    return pl.pallas_call(
        _t3_body,
        grid=(GRID,),
        in_specs=[_nc_spec(64), _nc_spec(16), _row_spec(1), _const_spec((B, 3))]
        + [_const_spec(c.shape) for c in consts],
        out_specs=pl.BlockSpec((B, 2), lambda i: (0, 0)),
        out_shape=jax.ShapeDtypeStruct((B, 2), jnp.float32),
        scratch_shapes=[pltpu.VMEM((B, HID), jnp.float32),
                        pltpu.VMEM((B, HID), jnp.float32),
                        pltpu.VMEM((1, B), jnp.float32)],
    )(raw2, den2, bat2, gf, *consts)


def _attn_proj(a):
    heads = a.shape[0]
    eye = jnp.eye(heads, dtype=jnp.float32)
    m = (eye[:, None, :] * a[:, :, None]).reshape(heads * HID, heads)
    return jnp.pad(m, ((0, 0), (0, 16 - heads)))


def kernel(x, graph_features, params, edge_index, batch):
    p = params
    loop = jnp.arange(N, dtype=edge_index.dtype)
    padv = jnp.zeros((EP - E_TOT,), edge_index.dtype)
    src_p = jnp.concatenate([edge_index[0], loop, padv])
    dst_p = jnp.concatenate([edge_index[1], loop, padv])
    x8 = jnp.pad(x, ((0, 0), (0, 5)))
    w0p = jnp.pad(p['W0'], ((0, 5), (0, 0)))
    bat2 = batch.reshape(N, 1)

    def vec(v):
        return jnp.reshape(v, (1, -1))

    h0c0, h0c1, h0c2, h0c3, es0, ed0 = _tc0(x8, w0p, _attn_proj(p['as0']),
                                            _attn_proj(p['ad0']))
    ext0, den0 = _sc_attention(es0, ed0, src_p, dst_p)
    raw0 = _sc_agg_chunks([h0c0, h0c1, h0c2, h0c3], ext0, src_p, dst_p, HEADS)
    t1 = _tc1(raw0, den0, vec(p['b0']), vec(p['g0']), vec(p['be0']),
              p['W1'], _attn_proj(p['as1']), _attn_proj(p['ad1']))
    zres, h1c, (es1, ed1) = t1[:4], t1[4:8], t1[8:]
    ext1, den1 = _sc_attention(es1, ed1, src_p, dst_p)
    raw1 = _sc_agg_chunks(list(h1c), ext1, src_p, dst_p, HEADS)
    h2, es2, ed2 = _tc2(raw1, den1, list(zres), vec(p['b1']), vec(p['g1']),
                        vec(p['be1']), p['W2'], _attn_proj(p['as2']),
                        _attn_proj(p['ad2']))
    ext2, den2 = _sc_attention(es2, ed2, src_p, dst_p)
    raw2 = _sc_agg_chunks([h2], ext2, src_p, dst_p, 1)
    return _tc3(raw2[0], den2, bat2, graph_features, p)


# speedup vs baseline: 38.2278x; 2.2136x over previous
import functools
import math

import jax
import jax.numpy as jnp
from jax import lax
from jax.experimental import pallas as pl
from jax.experimental.pallas import tpu as pltpu
from jax.experimental.pallas import tpu_sc as plsc

N = 10000
E = 160000
B = 32
HID = 64
HEADS = 8
D1 = HID * HEADS

NC, NS, LANES = 2, 16, 16
NW = NC * NS
KE = 128
NBLK = 44
EW = KE * NBLK
EP = NW * EW
E_TOT = E + N
NP = 10240
NROWS_T = NP // NS


def _att_body(zeros_hbm, es_hbm, ed_hbm, src3_hbm, dst3_hbm, extw_hbm, den_hbm,
              sidx, didx, esb0, edb0, esb1, edb1, extb0, extb1, den_sp,
              gs0, gd0, gs1, gd1):
    cid = lax.axis_index("c")
    sid = lax.axis_index("s")
    wid = cid * NS + sid
    rows0 = sid * NROWS_T
    pltpu.sync_copy(zeros_hbm.at[pl.ds(rows0, NROWS_T)], den_sp.at[pl.ds(rows0, NROWS_T)])
    pltpu.sync_copy(src3_hbm.at[wid], sidx)
    pltpu.sync_copy(dst3_hbm.at[wid], didx)
    plsc.subcore_barrier()
    lanes = lax.iota(jnp.int32, 16)
    esb = [esb0, esb1]
    edb = [edb0, edb1]
    extb = [extb0, extb1]
    gs = [gs0, gs1]
    gd = [gd0, gd1]

    def gathers(i, b):
        pltpu.async_copy(es_hbm.at[sidx.at[i]], esb[b], gs[b])
        pltpu.async_copy(ed_hbm.at[didx.at[i]], edb[b], gd[b])

    gathers(0, 0)

    def outer(i0, carry):
        for b in range(2):
            i = i0 + b

            @pl.when(i + 1 < NBLK)
            def _():
                gathers(i + 1, 1 - b)

            pltpu.make_async_copy(es_hbm.at[sidx.at[i]], esb[b], gs[b]).wait()
            pltpu.make_async_copy(ed_hbm.at[didx.at[i]], edb[b], gd[b]).wait()
            base = wid * EW + i * KE

            def edge(e, c2, b=b):
                v = esb[b][e, :] + edb[b][e, :]
                v = jnp.maximum(v, 0.2 * v)
                v = jnp.exp(v)
                v = jnp.where(base + e < E_TOT, v, jnp.zeros_like(v))
                edb[b][e, :] = v
                plsc.store_scatter(extb[b], [lanes * KE + e], v, mask=lanes < HEADS)
                return c2

            lax.fori_loop(0, KE, edge, 0, unroll=2)
            pltpu.sync_copy(edb[b], den_sp.at[didx.at[i]], add=True)
            pltpu.sync_copy(extb[b], extw_hbm.at[wid, i])
        return carry

    lax.fori_loop(0, NBLK // 2, lambda j, c: outer(j * 2, c), 0)
    plsc.subcore_barrier()
    pltpu.sync_copy(den_sp.at[pl.ds(rows0, NROWS_T)],
                    den_hbm.at[cid, pl.ds(rows0, NROWS_T)])


_att_call = functools.partial(
    pl.kernel,
    out_type=[jax.ShapeDtypeStruct((NW, NBLK, HEADS * KE), jnp.float32),
              jax.ShapeDtypeStruct((NC, NP, 16), jnp.float32)],
    mesh=plsc.VectorSubcoreMesh(core_axis_name="c", subcore_axis_name="s"),
    compiler_params=pltpu.CompilerParams(needs_layout_passes=False,
                                         use_tc_tiling_on_sc=False),
    scratch_types=[
        pltpu.VMEM((NBLK, KE), jnp.int32),
        pltpu.VMEM((NBLK, KE), jnp.int32),
        pltpu.VMEM((KE, 16), jnp.float32),
        pltpu.VMEM((KE, 16), jnp.float32),
        pltpu.VMEM((KE, 16), jnp.float32),
        pltpu.VMEM((KE, 16), jnp.float32),
        pltpu.VMEM((HEADS * KE,), jnp.float32),
        pltpu.VMEM((HEADS * KE,), jnp.float32),
        pltpu.VMEM_SHARED((NP, 16), jnp.float32),
        pltpu.SemaphoreType.DMA,
        pltpu.SemaphoreType.DMA,
        pltpu.SemaphoreType.DMA,
        pltpu.SemaphoreType.DMA,
    ],
)


@functools.lru_cache(maxsize=None)
def _make_agg(fc, nchunk, heads):
    hpc = fc // HID
    nq = fc // 16

    def body(*refs):
        harr = refs[:nchunk]
        extw_hbm, src3_hbm, dst3_hbm, zeros_hbm = refs[nchunk:nchunk + 4]
        outs = refs[nchunk + 4:2 * nchunk + 4]
        (sidx, didx, rows0b, rows1b, exa0, exa1, exb0, exb1, acc_sp,
         g0, g1, s0, s1) = refs[2 * nchunk + 4:]
        cid = lax.axis_index("c")
        sid = lax.axis_index("s")
        wid = cid * NS + sid
        rows0 = sid * NROWS_T
        rowsb = [rows0b, rows1b]
        exab = [exa0, exa1]
        exbb = [exb0, exb1]
        gsem = [g0, g1]
        ssem = [s0, s1]
        pltpu.sync_copy(src3_hbm.at[wid], sidx)
        pltpu.sync_copy(dst3_hbm.at[wid], didx)
        for ci in range(nchunk):
            ha = min(ci * hpc, heads - 1)
            hb = min(ci * hpc + 1, heads - 1)
            pltpu.sync_copy(zeros_hbm, acc_sp.at[pl.ds(rows0, NROWS_T)])
            plsc.subcore_barrier()

            def gather(i, b, ci=ci, ha=ha, hb=hb):
                pltpu.async_copy(harr[ci].at[sidx.at[i]], rowsb[b], gsem[b])
                pltpu.async_copy(extw_hbm.at[wid, i, pl.ds(ha * KE, KE)],
                                 exab[b], gsem[b])
                if hpc > 1:
                    pltpu.async_copy(extw_hbm.at[wid, i, pl.ds(hb * KE, KE)],
                                     exbb[b], gsem[b])

            def gwait(b, ci=ci, ha=ha, hb=hb):
                pltpu.make_async_copy(harr[ci].at[sidx.at[0]], rowsb[b],
                                      gsem[b]).wait()
                pltpu.make_async_copy(extw_hbm.at[wid, 0, pl.ds(ha * KE, KE)],
                                      exab[b], gsem[b]).wait()
                if hpc > 1:
                    pltpu.make_async_copy(extw_hbm.at[wid, 0, pl.ds(hb * KE, KE)],
                                          exbb[b], gsem[b]).wait()

            gather(0, 0)

            def outer(i0, carry, ci=ci):
                for b in range(2):
                    i = i0 + b

                    @pl.when(i + 1 < NBLK)
                    def _():
                        @pl.when(i >= 1)
                        def _():
                            pltpu.make_async_copy(
                                rowsb[1 - b], acc_sp.at[didx.at[0]],
                                ssem[1 - b]).wait()

                        gather(i + 1, 1 - b)

                    gwait(b)

                    def grp(g, c2, b=b):
                        e0 = g * 16
                        av = exab[b][pl.ds(e0, 16)]
                        bv = exbb[b][pl.ds(e0, 16)] if hpc > 1 else av
                        for j in range(16):
                            e = e0 + j
                            a = av[j]
                            for q in range(min(4, nq)):
                                rowsb[b][e, pl.ds(q * 16, 16)] = (
                                    rowsb[b][e, pl.ds(q * 16, 16)] * a)
                            if hpc > 1:
                                bb = bv[j]
                                for q in range(4, 8):
                                    rowsb[b][e, pl.ds(q * 16, 16)] = (
                                        rowsb[b][e, pl.ds(q * 16, 16)] * bb)
                        return c2

                    lax.fori_loop(0, KE // 16, grp, 0)
                    pltpu.async_copy(rowsb[b], acc_sp.at[didx.at[i]], ssem[b],
                                     add=True)
                return carry

            lax.fori_loop(0, NBLK // 2, lambda j, c: outer(j * 2, c), 0)
            for b in range(2):
                pltpu.make_async_copy(rowsb[b], acc_sp.at[didx.at[0]],
                                      ssem[b]).wait()
            plsc.subcore_barrier()
            pltpu.sync_copy(acc_sp.at[pl.ds(rows0, NROWS_T)],
                            outs[ci].at[cid, pl.ds(rows0, NROWS_T)])
            if ci + 1 < nchunk:
                plsc.subcore_barrier()

    call = functools.partial(
        pl.kernel,
        out_type=[jax.ShapeDtypeStruct((NC, NP, fc), jnp.float32)
                  for _ in range(nchunk)],
        mesh=plsc.VectorSubcoreMesh(core_axis_name="c", subcore_axis_name="s"),
        compiler_params=pltpu.CompilerParams(needs_layout_passes=False,
                                             use_tc_tiling_on_sc=False),
        scratch_types=[
            pltpu.VMEM((NBLK, KE), jnp.int32),
            pltpu.VMEM((NBLK, KE), jnp.int32),
            pltpu.VMEM((KE, fc), jnp.float32),
            pltpu.VMEM((KE, fc), jnp.float32),
            pltpu.VMEM((KE,), jnp.float32),
            pltpu.VMEM((KE,), jnp.float32),
            pltpu.VMEM((KE,), jnp.float32),
            pltpu.VMEM((KE,), jnp.float32),
            pltpu.VMEM_SHARED((NP, fc), jnp.float32),
            pltpu.SemaphoreType.DMA,
            pltpu.SemaphoreType.DMA,
            pltpu.SemaphoreType.DMA,
            pltpu.SemaphoreType.DMA,
        ],
    )
    return call(body)


def _sc_agg_chunks(harr, extw, src3, dst3, heads):
    fc = harr[0].shape[1]
    agg = _make_agg(fc, len(harr), heads)
    zeros = jnp.zeros((NROWS_T, fc), jnp.float32)
    outs = agg(*harr, extw, src3, dst3, zeros)
    return list(outs) if isinstance(outs, (list, tuple)) else [outs]


def _sc_attention(esp, edp, src3, dst3):
    zeros = jnp.zeros((NP, 16), jnp.float32)
    extw, den = _att_call(_att_body)(zeros, esp, edp, src3, dst3)
    return extw, den


RB = 1000
GRID = N // RB
PREC = jax.lax.Precision.DEFAULT
BNS = 1.0 / math.sqrt(1.0 + 1e-5)


def _dot(a, b):
    return jax.lax.dot_general(a, b, (((1,), (0,)), ((), ())),
                               precision=PREC, preferred_element_type=jnp.float32)


def _proj_outs(h, as_ref, ad_ref, houts, es_ref, ed_ref):
    nck = len(houts)
    for ci in range(nck):
        houts[ci][...] = h[:, ci * 128:(ci + 1) * 128] if nck > 1 else h
    heads = as_ref.shape[0]
    asm, adm = as_ref[...], ad_ref[...]
    rows = h.shape[0]
    zpad = jnp.zeros((rows, 16 - heads), jnp.float32)
    es_parts, ed_parts = [], []
    for hd in range(heads):
        hs = h[:, hd * HID:(hd + 1) * HID]
        es_parts.append(jnp.sum(hs * asm[hd:hd + 1, :], axis=1, keepdims=True))
        ed_parts.append(jnp.sum(hs * adm[hd:hd + 1, :], axis=1, keepdims=True))
    es_ref[...] = jnp.concatenate(es_parts + [zpad], axis=1)
    ed_ref[...] = jnp.concatenate(ed_parts + [zpad], axis=1)


def _t0_body(x_ref, w_ref, as_ref, ad_ref, h0, h1, h2, h3, es_ref, ed_ref):
    h = _dot(x_ref[...], w_ref[...])
    _proj_outs(h, as_ref, ad_ref, [h0, h1, h2, h3], es_ref, ed_ref)


def _assemble(raws, den_ref, bv, gv, bev):
    den = den_ref[0] + den_ref[1]
    zs = []
    for ci in range(4):
        r = raws[ci][0] + raws[ci][1]
        da = den[:, 2 * ci:2 * ci + 1]
        db = den[:, 2 * ci + 1:2 * ci + 2]
        z = jnp.concatenate([r[:, :64] / da, r[:, 64:] / db], axis=1)
        z = z + bv[:, ci * 128:(ci + 1) * 128]
        z = z * (gv[:, ci * 128:(ci + 1) * 128] * BNS) + bev[:, ci * 128:(ci + 1) * 128]
        zs.append(jnp.maximum(z, 0.0))
    return jnp.concatenate(zs, axis=1)


def _t1_body(r0, r1, r2, r3, den_ref, bv, gv, bev, w_ref, as_ref, ad_ref,
             z0, z1, z2, z3, h0, h1, h2, h3, es_ref, ed_ref):
    z = _assemble([r0, r1, r2, r3], den_ref, bv[...], gv[...], bev[...])
    for ci, zr in enumerate([z0, z1, z2, z3]):
        zr[...] = z[:, ci * 128:(ci + 1) * 128]
    h = _dot(z, w_ref[...])
    _proj_outs(h, as_ref, ad_ref, [h0, h1, h2, h3], es_ref, ed_ref)


def _t2_body(r0, r1, r2, r3, den_ref, zr0, zr1, zr2, zr3, bv, gv, bev,
             w_ref, as_ref, ad_ref, h2out, es_ref, ed_ref):
    z = _assemble([r0, r1, r2, r3], den_ref, bv[...], gv[...], bev[...])
    res = jnp.concatenate([zr0[...], zr1[...], zr2[...], zr3[...]], axis=1)
    h = _dot(res + z, w_ref[...])
    _proj_outs(h, as_ref, ad_ref, [h2out], es_ref, ed_ref)


def _t3_body(r_ref, den_ref, bat_ref, gf_ref, b2v, g2v, be2v,
             wg, bgv, wr1, br1, gr1, ber1, wr2, br2, gr2, ber2, wr3, br3,
             out_ref, ssum, smax, scnt):
    pid = pl.program_id(0)

    @pl.when(pid == 0)
    def _():
        ssum[...] = jnp.zeros_like(ssum)
        smax[...] = jnp.zeros_like(smax)
        scnt[...] = jnp.zeros_like(scnt)

    den = den_ref[0, :, :1] + den_ref[1, :, :1]
    h = (r_ref[0] + r_ref[1]) / den + b2v[...]
    h = jnp.maximum(h * (g2v[...] * BNS) + be2v[...], 0.0)
    bat = bat_ref[...]
    mask = (bat == jax.lax.broadcasted_iota(jnp.int32, (RB, B), 1)).astype(jnp.float32)
    ssum[...] += jax.lax.dot_general(mask, h, (((0,), (0,)), ((), ())),
                                     precision=PREC, preferred_element_type=jnp.float32)
    scnt[...] += jnp.sum(mask, axis=0, keepdims=True)
    for b in range(B):
        cand = jnp.max(h * mask[:, b:b + 1], axis=0, keepdims=True)
        smax[pl.ds(b, 1), :] = jnp.maximum(smax[pl.ds(b, 1), :], cand)

    @pl.when(pid == GRID - 1)
    def _():
        cnt = jnp.maximum(scnt[...], 1.0).reshape(B, 1)
        mean = ssum[...] / cnt
        gfv = jnp.maximum(_dot(gf_ref[...], wg[...]) + bgv[...], 0.0)
        c = jnp.concatenate([mean, smax[...], gfv], axis=1)
        zz = _dot(c, wr1[...]) + br1[...]
        zz = jnp.maximum(zz * (gr1[...] * BNS) + ber1[...], 0.0)
        zz = _dot(zz, wr2[...]) + br2[...]
        zz = jnp.maximum(zz * (gr2[...] * BNS) + ber2[...], 0.0)
        out_ref[...] = _dot(zz, wr3[...]) + br3[...]


def _row_spec(w):
    return pl.BlockSpec((RB, w), lambda i: (i, 0))


def _const_spec(shape):
    nd = len(shape)
    return pl.BlockSpec(shape, lambda i, _n=nd: (0,) * _n)


def _nc_spec(w):
    return pl.BlockSpec((NC, RB, w), lambda i: (0, i, 0))


def _tc0(x8, w0p, as0p, ad0p):
    return pl.pallas_call(
        _t0_body,
        grid=(GRID,),
        in_specs=[_row_spec(8), _const_spec((8, D1)), _const_spec((HEADS, HID)),
                  _const_spec((HEADS, HID))],
        out_specs=[_row_spec(128)] * 4 + [_row_spec(16)] * 2,
        out_shape=[jax.ShapeDtypeStruct((N, 128), jnp.float32)] * 4
        + [jax.ShapeDtypeStruct((N, 16), jnp.float32)] * 2,
    )(x8, w0p, as0p, ad0p)


def _tc1(raws, den, bv, gv, bev, w, asp, adp):
    return pl.pallas_call(
        _t1_body,
        grid=(GRID,),
        in_specs=[_nc_spec(128)] * 4 + [_nc_spec(16)]
        + [_const_spec((1, D1))] * 3
        + [_const_spec((D1, D1)), _const_spec((HEADS, HID)), _const_spec((HEADS, HID))],
        out_specs=[_row_spec(128)] * 8 + [_row_spec(16)] * 2,
        out_shape=[jax.ShapeDtypeStruct((N, 128), jnp.float32)] * 8
        + [jax.ShapeDtypeStruct((N, 16), jnp.float32)] * 2,
    )(*raws, den, bv, gv, bev, w, asp, adp)


def _tc2(raws, den, zres, bv, gv, bev, w, asp, adp):
    return pl.pallas_call(
        _t2_body,
        grid=(GRID,),
        in_specs=[_nc_spec(128)] * 4 + [_nc_spec(16)] + [_row_spec(128)] * 4
        + [_const_spec((1, D1))] * 3
        + [_const_spec((D1, HID)), _const_spec((1, HID)), _const_spec((1, HID))],
        out_specs=[_row_spec(64)] + [_row_spec(16)] * 2,
        out_shape=[jax.ShapeDtypeStruct((N, 64), jnp.float32)]
        + [jax.ShapeDtypeStruct((N, 16), jnp.float32)] * 2,
    )(*raws, den, *zres, bv, gv, bev, w, asp, adp)


def _tc3(raw2, den2, bat2, gf, p):
    consts = [jnp.reshape(p['b2'], (1, HID)), jnp.reshape(p['g2'], (1, HID)),
              jnp.reshape(p['be2'], (1, HID)), p['Wg'],
              jnp.reshape(p['bg'], (1, HID // 2)), p['Wr1'],
              jnp.reshape(p['br1'], (1, HID)), jnp.reshape(p['gr1'], (1, HID)),
              jnp.reshape(p['ber1'], (1, HID)), p['Wr2'],
              jnp.reshape(p['br2'], (1, HID // 2)), jnp.reshape(p['gr2'], (1, HID // 2)),
              jnp.reshape(p['ber2'], (1, HID // 2)), p['Wr3'],
              jnp.reshape(p['br3'], (1, 2))]
    return pl.pallas_call(
        _t3_body,
        grid=(GRID,),
        in_specs=[_nc_spec(64), _nc_spec(16), _row_spec(1), _const_spec((B, 3))]
        + [_const_spec(c.shape) for c in consts],
        out_specs=pl.BlockSpec((B, 2), lambda i: (0, 0)),
        out_shape=jax.ShapeDtypeStruct((B, 2), jnp.float32),
        scratch_shapes=[pltpu.VMEM((B, HID), jnp.float32),
                        pltpu.VMEM((B, HID), jnp.float32),
                        pltpu.VMEM((1, B), jnp.float32)],
    )(raw2, den2, bat2, gf, *consts)


def _attn_proj(a):
    heads = a.shape[0]
    eye = jnp.eye(heads, dtype=jnp.float32)
    m = (eye[:, None, :] * a[:, :, None]).reshape(heads * HID, heads)
    return jnp.pad(m, ((0, 0), (0, 16 - heads)))


_JNP_TAIL = False
_JNP_DENSE = False


def _jnp_dense_path(x, graph_features, p, src3, dst3, batch):
    def pad16(a):
        return jnp.pad(a, ((0, 0), (0, 16 - a.shape[1])))

    def gat(hin, W, a_s, a_d, heads):
        h = hin @ W
        hr = h.reshape(N, heads, HID)
        es = pad16((hr * a_s[None]).sum(-1))
        ed = pad16((hr * a_d[None]).sum(-1))
        extw, den = _sc_attention(es, ed, src3, dst3)
        harr = [h[:, i * 128:(i + 1) * 128] for i in range(max(1, heads * HID // 128))]
        if heads * HID < 128:
            harr = [h]
        raws = _sc_agg_chunks(harr, extw, src3, dst3, heads)
        raw = jnp.concatenate([o[0, :N] + o[1, :N] for o in raws], axis=1)
        denf = den[0, :N, :heads] + den[1, :N, :heads]
        out = raw.reshape(N, heads, HID) / denf[:, :, None]
        return out.reshape(N, heads * HID)

    z0 = jax.nn.relu((gat(x, p['W0'], p['as0'], p['ad0'], HEADS) + p['b0'])
                     * BNS * p['g0'] + p['be0'])
    z1 = jax.nn.relu((gat(z0, p['W1'], p['as1'], p['ad1'], HEADS) + p['b1'])
                     * BNS * p['g1'] + p['be1'])
    hin2 = z0 + z1
    h3 = gat(hin2, p['W2'], p['as2'], p['ad2'], 1) + p['b2']
    h = jax.nn.relu(h3 * BNS * p['g2'] + p['be2'])
    counts = jax.ops.segment_sum(jnp.ones((N, 1), jnp.float32), batch, num_segments=B)
    mean_pool = jax.ops.segment_sum(h, batch, num_segments=B) / jnp.maximum(counts, 1.0)
    max_pool = jax.ops.segment_max(h, batch, num_segments=B)
    gf = jax.nn.relu(graph_features @ p['Wg'] + p['bg'])
    c = jnp.concatenate([mean_pool, max_pool, gf], axis=1)
    z = jax.nn.relu((c @ p['Wr1'] + p['br1']) * BNS * p['gr1'] + p['ber1'])
    z = jax.nn.relu((z @ p['Wr2'] + p['br2']) * BNS * p['gr2'] + p['ber2'])
    return z @ p['Wr3'] + p['br3']


def kernel(x, graph_features, params, edge_index, batch):
    p = params
    loop = jnp.arange(N, dtype=edge_index.dtype)
    padv = jnp.arange(EP - E_TOT, dtype=edge_index.dtype) % N
    src_p = jnp.concatenate([edge_index[0], loop, padv]).reshape(NW, NBLK, KE)
    dst_p = jnp.concatenate([edge_index[1], loop, padv]).reshape(NW, NBLK, KE)
    x8 = jnp.pad(x, ((0, 0), (0, 5)))
    w0p = jnp.pad(p['W0'], ((0, 5), (0, 0)))
    bat2 = batch.reshape(N, 1)

    def vec(v):
        return jnp.reshape(v, (1, -1))

    if _JNP_DENSE:
        return _jnp_dense_path(x, graph_features, p, src_p, dst_p, batch)
    h0c0, h0c1, h0c2, h0c3, es0, ed0 = _tc0(x8, w0p, p['as0'], p['ad0'])
    ext0, den0 = _sc_attention(es0, ed0, src_p, dst_p)
    raw0 = _sc_agg_chunks([h0c0, h0c1, h0c2, h0c3], ext0, src_p, dst_p, HEADS)
    t1 = _tc1(raw0, den0, vec(p['b0']), vec(p['g0']), vec(p['be0']),
              p['W1'], p['as1'], p['ad1'])
    zres, h1c, (es1, ed1) = t1[:4], t1[4:8], t1[8:]
    ext1, den1 = _sc_attention(es1, ed1, src_p, dst_p)
    raw1 = _sc_agg_chunks(list(h1c), ext1, src_p, dst_p, HEADS)
    h2, es2, ed2 = _tc2(raw1, den1, list(zres), vec(p['b1']), vec(p['g1']),
                        vec(p['be1']), p['W2'], p['as2'], p['ad2'])
    ext2, den2 = _sc_attention(es2, ed2, src_p, dst_p)
    raw2 = _sc_agg_chunks([h2], ext2, src_p, dst_p, 1)
    if _JNP_TAIL:
        den = den2[0, :N, :1] + den2[1, :N, :1]
        h3 = (raw2[0][0, :N] + raw2[0][1, :N]) / den + p['b2']
        h = jax.nn.relu(h3 * BNS * p['g2'] + p['be2'])
        counts = jax.ops.segment_sum(jnp.ones((N, 1), jnp.float32), batch, num_segments=B)
        mean_pool = jax.ops.segment_sum(h, batch, num_segments=B) / jnp.maximum(counts, 1.0)
        max_pool = jax.ops.segment_max(h, batch, num_segments=B)
        gf = jax.nn.relu(graph_features @ p['Wg'] + p['bg'])
        c = jnp.concatenate([mean_pool, max_pool, gf], axis=1)
        z = jax.nn.relu((c @ p['Wr1'] + p['br1']) * BNS * p['gr1'] + p['ber1'])
        z = jax.nn.relu((z @ p['Wr2'] + p['br2']) * BNS * p['gr2'] + p['ber2'])
        return z @ p['Wr3'] + p['br3']
    return _tc3(raw2[0], den2, bat2, graph_features, p)


# speedup vs baseline: 38.5532x; 1.0085x over previous
import functools
import math

import jax
import jax.numpy as jnp
from jax import lax
from jax.experimental import pallas as pl
from jax.experimental.pallas import tpu as pltpu
from jax.experimental.pallas import tpu_sc as plsc

N = 10000
E = 160000
B = 32
HID = 64
HEADS = 8
D1 = HID * HEADS

NC, NS, LANES = 2, 16, 16
NW = NC * NS
KE = 128
NBLK = 44
EW = KE * NBLK
EP = NW * EW
E_TOT = E + N
NP = 10240
NROWS_T = NP // NS


def _att_body(zeros_hbm, es_hbm, ed_hbm, src3_hbm, dst3_hbm, extw_hbm, den_hbm,
              sidx, didx, esb0, edb0, esb1, edb1, extb0, extb1, den_sp,
              gs0, gd0, gs1, gd1, ds0, ds1, xs0, xs1):
    cid = lax.axis_index("c")
    sid = lax.axis_index("s")
    wid = cid * NS + sid
    rows0 = sid * NROWS_T
    pltpu.sync_copy(zeros_hbm.at[pl.ds(rows0, NROWS_T)], den_sp.at[pl.ds(rows0, NROWS_T)])
    pltpu.sync_copy(src3_hbm.at[wid], sidx)
    pltpu.sync_copy(dst3_hbm.at[wid], didx)
    plsc.subcore_barrier()
    lanes = lax.iota(jnp.int32, 16)
    esb = [esb0, esb1]
    edb = [edb0, edb1]
    extb = [extb0, extb1]
    gs = [gs0, gs1]
    gd = [gd0, gd1]
    dsem = [ds0, ds1]
    xsem = [xs0, xs1]

    def gathers(i, b):
        pltpu.async_copy(es_hbm.at[sidx.at[i]], esb[b], gs[b])
        pltpu.async_copy(ed_hbm.at[didx.at[i]], edb[b], gd[b])

    gathers(0, 0)

    def outer(i0, carry):
        for b in range(2):
            i = i0 + b

            @pl.when(i + 1 < NBLK)
            def _():
                @pl.when(i >= 1)
                def _():
                    pltpu.make_async_copy(edb[1 - b], den_sp.at[didx.at[0]],
                                          dsem[1 - b]).wait()
                    pltpu.make_async_copy(extb[1 - b], extw_hbm.at[wid, 0],
                                          xsem[1 - b]).wait()

                gathers(i + 1, 1 - b)

            pltpu.make_async_copy(es_hbm.at[sidx.at[i]], esb[b], gs[b]).wait()
            pltpu.make_async_copy(ed_hbm.at[didx.at[i]], edb[b], gd[b]).wait()
            base = wid * EW + i * KE

            def edge(e, c2, b=b):
                v = esb[b][e, :] + edb[b][e, :]
                v = jnp.maximum(v, 0.2 * v)
                v = jnp.exp(v)
                v = jnp.where(base + e < E_TOT, v, jnp.zeros_like(v))
                edb[b][e, :] = v
                plsc.store_scatter(extb[b], [lanes * KE + e], v, mask=lanes < HEADS)
                return c2

            lax.fori_loop(0, KE, edge, 0, unroll=2)
            pltpu.async_copy(edb[b], den_sp.at[didx.at[i]], dsem[b], add=True)
            pltpu.async_copy(extb[b], extw_hbm.at[wid, i], xsem[b])
        return carry

    lax.fori_loop(0, NBLK // 2, lambda j, c: outer(j * 2, c), 0)
    for b in range(2):
        pltpu.make_async_copy(edb[b], den_sp.at[didx.at[0]], dsem[b]).wait()
        pltpu.make_async_copy(extb[b], extw_hbm.at[wid, 0], xsem[b]).wait()
    plsc.subcore_barrier()
    pltpu.sync_copy(den_sp.at[pl.ds(rows0, NROWS_T)],
                    den_hbm.at[cid, pl.ds(rows0, NROWS_T)])


_att_call = functools.partial(
    pl.kernel,
    out_type=[jax.ShapeDtypeStruct((NW, NBLK, HEADS * KE), jnp.float32),
              jax.ShapeDtypeStruct((NC, NP, 16), jnp.float32)],
    mesh=plsc.VectorSubcoreMesh(core_axis_name="c", subcore_axis_name="s"),
    compiler_params=pltpu.CompilerParams(needs_layout_passes=False,
                                         use_tc_tiling_on_sc=False),
    scratch_types=[
        pltpu.VMEM((NBLK, KE), jnp.int32),
        pltpu.VMEM((NBLK, KE), jnp.int32),
        pltpu.VMEM((KE, 16), jnp.float32),
        pltpu.VMEM((KE, 16), jnp.float32),
        pltpu.VMEM((KE, 16), jnp.float32),
        pltpu.VMEM((KE, 16), jnp.float32),
        pltpu.VMEM((HEADS * KE,), jnp.float32),
        pltpu.VMEM((HEADS * KE,), jnp.float32),
        pltpu.VMEM_SHARED((NP, 16), jnp.float32),
        pltpu.SemaphoreType.DMA,
        pltpu.SemaphoreType.DMA,
        pltpu.SemaphoreType.DMA,
        pltpu.SemaphoreType.DMA,
        pltpu.SemaphoreType.DMA,
        pltpu.SemaphoreType.DMA,
        pltpu.SemaphoreType.DMA,
        pltpu.SemaphoreType.DMA,
    ],
)


@functools.lru_cache(maxsize=None)
def _make_agg(fc, nchunk, heads):
    hpc = fc // HID
    nq = fc // 16

    def body(*refs):
        harr = refs[:nchunk]
        extw_hbm, src3_hbm, dst3_hbm, zeros_hbm = refs[nchunk:nchunk + 4]
        outs = refs[nchunk + 4:2 * nchunk + 4]
        (sidx, didx, rows0b, rows1b, exa0, exa1, exb0, exb1, acc_sp,
         g0, g1, s0, s1) = refs[2 * nchunk + 4:]
        cid = lax.axis_index("c")
        sid = lax.axis_index("s")
        wid = cid * NS + sid
        rows0 = sid * NROWS_T
        rowsb = [rows0b, rows1b]
        exab = [exa0, exa1]
        exbb = [exb0, exb1]
        gsem = [g0, g1]
        ssem = [s0, s1]
        pltpu.sync_copy(src3_hbm.at[wid], sidx)
        pltpu.sync_copy(dst3_hbm.at[wid], didx)
        for ci in range(nchunk):
            ha = min(ci * hpc, heads - 1)
            hb = min(ci * hpc + 1, heads - 1)
            pltpu.sync_copy(zeros_hbm, acc_sp.at[pl.ds(rows0, NROWS_T)])
            plsc.subcore_barrier()

            def gather(i, b, ci=ci, ha=ha, hb=hb):
                pltpu.async_copy(harr[ci].at[sidx.at[i]], rowsb[b], gsem[b])
                pltpu.async_copy(extw_hbm.at[wid, i, pl.ds(ha * KE, KE)],
                                 exab[b], gsem[b])
                if hpc > 1:
                    pltpu.async_copy(extw_hbm.at[wid, i, pl.ds(hb * KE, KE)],
                                     exbb[b], gsem[b])

            def gwait(b, ci=ci, ha=ha, hb=hb):
                pltpu.make_async_copy(harr[ci].at[sidx.at[0]], rowsb[b],
                                      gsem[b]).wait()
                pltpu.make_async_copy(extw_hbm.at[wid, 0, pl.ds(ha * KE, KE)],
                                      exab[b], gsem[b]).wait()
                if hpc > 1:
                    pltpu.make_async_copy(extw_hbm.at[wid, 0, pl.ds(hb * KE, KE)],
                                          exbb[b], gsem[b]).wait()

            gather(0, 0)

            def outer(i0, carry, ci=ci):
                for b in range(2):
                    i = i0 + b

                    @pl.when(i + 1 < NBLK)
                    def _():
                        @pl.when(i >= 1)
                        def _():
                            pltpu.make_async_copy(
                                rowsb[1 - b], acc_sp.at[didx.at[0]],
                                ssem[1 - b]).wait()

                        gather(i + 1, 1 - b)

                    gwait(b)

                    def grp(g, c2, b=b):
                        e0 = g * 16
                        av = exab[b][pl.ds(e0, 16)]
                        bv = exbb[b][pl.ds(e0, 16)] if hpc > 1 else av
                        for j in range(16):
                            e = e0 + j
                            a = av[j]
                            for q in range(min(4, nq)):
                                rowsb[b][e, pl.ds(q * 16, 16)] = (
                                    rowsb[b][e, pl.ds(q * 16, 16)] * a)
                            if hpc > 1:
                                bb = bv[j]
                                for q in range(4, 8):
                                    rowsb[b][e, pl.ds(q * 16, 16)] = (
                                        rowsb[b][e, pl.ds(q * 16, 16)] * bb)
                        return c2

                    lax.fori_loop(0, KE // 16, grp, 0)
                    pltpu.async_copy(rowsb[b], acc_sp.at[didx.at[i]], ssem[b],
                                     add=True)
                return carry

            lax.fori_loop(0, NBLK // 2, lambda j, c: outer(j * 2, c), 0)
            for b in range(2):
                pltpu.make_async_copy(rowsb[b], acc_sp.at[didx.at[0]],
                                      ssem[b]).wait()
            plsc.subcore_barrier()
            pltpu.sync_copy(acc_sp.at[pl.ds(rows0, NROWS_T)],
                            outs[ci].at[cid, pl.ds(rows0, NROWS_T)])
            if ci + 1 < nchunk:
                plsc.subcore_barrier()

    call = functools.partial(
        pl.kernel,
        out_type=[jax.ShapeDtypeStruct((NC, NP, fc), jnp.float32)
                  for _ in range(nchunk)],
        mesh=plsc.VectorSubcoreMesh(core_axis_name="c", subcore_axis_name="s"),
        compiler_params=pltpu.CompilerParams(needs_layout_passes=False,
                                             use_tc_tiling_on_sc=False),
        scratch_types=[
            pltpu.VMEM((NBLK, KE), jnp.int32),
            pltpu.VMEM((NBLK, KE), jnp.int32),
            pltpu.VMEM((KE, fc), jnp.float32),
            pltpu.VMEM((KE, fc), jnp.float32),
            pltpu.VMEM((KE,), jnp.float32),
            pltpu.VMEM((KE,), jnp.float32),
            pltpu.VMEM((KE,), jnp.float32),
            pltpu.VMEM((KE,), jnp.float32),
            pltpu.VMEM_SHARED((NP, fc), jnp.float32),
            pltpu.SemaphoreType.DMA,
            pltpu.SemaphoreType.DMA,
            pltpu.SemaphoreType.DMA,
            pltpu.SemaphoreType.DMA,
        ],
    )
    return call(body)


def _sc_agg_chunks(harr, extw, src3, dst3, heads):
    fc = harr[0].shape[1]
    agg = _make_agg(fc, len(harr), heads)
    zeros = jnp.zeros((NROWS_T, fc), jnp.float32)
    outs = agg(*harr, extw, src3, dst3, zeros)
    return list(outs) if isinstance(outs, (list, tuple)) else [outs]


def _sc_attention(esp, edp, src3, dst3):
    zeros = jnp.zeros((NP, 16), jnp.float32)
    extw, den = _att_call(_att_body)(zeros, esp, edp, src3, dst3)
    return extw, den


RB = 1000
GRID = N // RB
PREC = jax.lax.Precision.DEFAULT
BNS = 1.0 / math.sqrt(1.0 + 1e-5)


def _dot(a, b):
    return jax.lax.dot_general(a, b, (((1,), (0,)), ((), ())),
                               precision=PREC, preferred_element_type=jnp.float32)


def _proj_outs(h, as_ref, ad_ref, houts, es_ref, ed_ref):
    nck = len(houts)
    for ci in range(nck):
        houts[ci][...] = h[:, ci * 128:(ci + 1) * 128] if nck > 1 else h
    heads = as_ref.shape[0]
    asm, adm = as_ref[...], ad_ref[...]
    rows = h.shape[0]
    zpad = jnp.zeros((rows, 16 - heads), jnp.float32)
    es_parts, ed_parts = [], []
    for hd in range(heads):
        hs = h[:, hd * HID:(hd + 1) * HID]
        es_parts.append(jnp.sum(hs * asm[hd:hd + 1, :], axis=1, keepdims=True))
        ed_parts.append(jnp.sum(hs * adm[hd:hd + 1, :], axis=1, keepdims=True))
    es_ref[...] = jnp.concatenate(es_parts + [zpad], axis=1)
    ed_ref[...] = jnp.concatenate(ed_parts + [zpad], axis=1)


def _t0_body(x_ref, w_ref, as_ref, ad_ref, h0, h1, h2, h3, es_ref, ed_ref):
    h = _dot(x_ref[...], w_ref[...])
    _proj_outs(h, as_ref, ad_ref, [h0, h1, h2, h3], es_ref, ed_ref)


def _assemble(raws, den_ref, bv, gv, bev):
    den = den_ref[0] + den_ref[1]
    zs = []
    for ci in range(4):
        r = raws[ci][0] + raws[ci][1]
        da = den[:, 2 * ci:2 * ci + 1]
        db = den[:, 2 * ci + 1:2 * ci + 2]
        z = jnp.concatenate([r[:, :64] / da, r[:, 64:] / db], axis=1)
        z = z + bv[:, ci * 128:(ci + 1) * 128]
        z = z * (gv[:, ci * 128:(ci + 1) * 128] * BNS) + bev[:, ci * 128:(ci + 1) * 128]
        zs.append(jnp.maximum(z, 0.0))
    return jnp.concatenate(zs, axis=1)


def _t1_body(r0, r1, r2, r3, den_ref, bv, gv, bev, w_ref, as_ref, ad_ref,
             z0, z1, z2, z3, h0, h1, h2, h3, es_ref, ed_ref):
    z = _assemble([r0, r1, r2, r3], den_ref, bv[...], gv[...], bev[...])
    for ci, zr in enumerate([z0, z1, z2, z3]):
        zr[...] = z[:, ci * 128:(ci + 1) * 128]
    h = _dot(z, w_ref[...])
    _proj_outs(h, as_ref, ad_ref, [h0, h1, h2, h3], es_ref, ed_ref)


def _t2_body(r0, r1, r2, r3, den_ref, zr0, zr1, zr2, zr3, bv, gv, bev,
             w_ref, as_ref, ad_ref, h2out, es_ref, ed_ref):
    z = _assemble([r0, r1, r2, r3], den_ref, bv[...], gv[...], bev[...])
    res = jnp.concatenate([zr0[...], zr1[...], zr2[...], zr3[...]], axis=1)
    h = _dot(res + z, w_ref[...])
    _proj_outs(h, as_ref, ad_ref, [h2out], es_ref, ed_ref)


def _t3_body(r_ref, den_ref, bat_ref, gf_ref, b2v, g2v, be2v,
             wg, bgv, wr1, br1, gr1, ber1, wr2, br2, gr2, ber2, wr3, br3,
             out_ref, ssum, smax, scnt):
    pid = pl.program_id(0)

    @pl.when(pid == 0)
    def _():
        ssum[...] = jnp.zeros_like(ssum)
        smax[...] = jnp.zeros_like(smax)
        scnt[...] = jnp.zeros_like(scnt)

    den = den_ref[0, :, :1] + den_ref[1, :, :1]
    h = (r_ref[0] + r_ref[1]) / den + b2v[...]
    h = jnp.maximum(h * (g2v[...] * BNS) + be2v[...], 0.0)
    bat = bat_ref[...]
    mask = (bat == jax.lax.broadcasted_iota(jnp.int32, (RB, B), 1)).astype(jnp.float32)
    ssum[...] += jax.lax.dot_general(mask, h, (((0,), (0,)), ((), ())),
                                     precision=PREC, preferred_element_type=jnp.float32)
    scnt[...] += jnp.sum(mask, axis=0, keepdims=True)
    for b in range(B):
        cand = jnp.max(h * mask[:, b:b + 1], axis=0, keepdims=True)
        smax[pl.ds(b, 1), :] = jnp.maximum(smax[pl.ds(b, 1), :], cand)

    @pl.when(pid == GRID - 1)
    def _():
        cnt = jnp.maximum(scnt[...], 1.0).reshape(B, 1)
        mean = ssum[...] / cnt
        gfv = jnp.maximum(_dot(gf_ref[...], wg[...]) + bgv[...], 0.0)
        c = jnp.concatenate([mean, smax[...], gfv], axis=1)
        zz = _dot(c, wr1[...]) + br1[...]
        zz = jnp.maximum(zz * (gr1[...] * BNS) + ber1[...], 0.0)
        zz = _dot(zz, wr2[...]) + br2[...]
        zz = jnp.maximum(zz * (gr2[...] * BNS) + ber2[...], 0.0)
        out_ref[...] = _dot(zz, wr3[...]) + br3[...]


def _row_spec(w):
    return pl.BlockSpec((RB, w), lambda i: (i, 0))


def _const_spec(shape):
    nd = len(shape)
    return pl.BlockSpec(shape, lambda i, _n=nd: (0,) * _n)


def _nc_spec(w):
    return pl.BlockSpec((NC, RB, w), lambda i: (0, i, 0))


def _tc0(x8, w0p, as0p, ad0p):
    return pl.pallas_call(
        _t0_body,
        grid=(GRID,),
        in_specs=[_row_spec(8), _const_spec((8, D1)), _const_spec((HEADS, HID)),
                  _const_spec((HEADS, HID))],
        out_specs=[_row_spec(128)] * 4 + [_row_spec(16)] * 2,
        out_shape=[jax.ShapeDtypeStruct((N, 128), jnp.float32)] * 4
        + [jax.ShapeDtypeStruct((N, 16), jnp.float32)] * 2,
    )(x8, w0p, as0p, ad0p)


def _tc1(raws, den, bv, gv, bev, w, asp, adp):
    return pl.pallas_call(
        _t1_body,
        grid=(GRID,),
        in_specs=[_nc_spec(128)] * 4 + [_nc_spec(16)]
        + [_const_spec((1, D1))] * 3
        + [_const_spec((D1, D1)), _const_spec((HEADS, HID)), _const_spec((HEADS, HID))],
        out_specs=[_row_spec(128)] * 8 + [_row_spec(16)] * 2,
        out_shape=[jax.ShapeDtypeStruct((N, 128), jnp.float32)] * 8
        + [jax.ShapeDtypeStruct((N, 16), jnp.float32)] * 2,
    )(*raws, den, bv, gv, bev, w, asp, adp)


def _tc2(raws, den, zres, bv, gv, bev, w, asp, adp):
    return pl.pallas_call(
        _t2_body,
        grid=(GRID,),
        in_specs=[_nc_spec(128)] * 4 + [_nc_spec(16)] + [_row_spec(128)] * 4
        + [_const_spec((1, D1))] * 3
        + [_const_spec((D1, HID)), _const_spec((1, HID)), _const_spec((1, HID))],
        out_specs=[_row_spec(64)] + [_row_spec(16)] * 2,
        out_shape=[jax.ShapeDtypeStruct((N, 64), jnp.float32)]
        + [jax.ShapeDtypeStruct((N, 16), jnp.float32)] * 2,
    )(*raws, den, *zres, bv, gv, bev, w, asp, adp)


def _tc3(raw2, den2, bat2, gf, p):
    consts = [jnp.reshape(p['b2'], (1, HID)), jnp.reshape(p['g2'], (1, HID)),
              jnp.reshape(p['be2'], (1, HID)), p['Wg'],
              jnp.reshape(p['bg'], (1, HID // 2)), p['Wr1'],
              jnp.reshape(p['br1'], (1, HID)), jnp.reshape(p['gr1'], (1, HID)),
              jnp.reshape(p['ber1'], (1, HID)), p['Wr2'],
              jnp.reshape(p['br2'], (1, HID // 2)), jnp.reshape(p['gr2'], (1, HID // 2)),
              jnp.reshape(p['ber2'], (1, HID // 2)), p['Wr3'],
              jnp.reshape(p['br3'], (1, 2))]
    return pl.pallas_call(
        _t3_body,
        grid=(GRID,),
        in_specs=[_nc_spec(64), _nc_spec(16), _row_spec(1), _const_spec((B, 3))]
        + [_const_spec(c.shape) for c in consts],
        out_specs=pl.BlockSpec((B, 2), lambda i: (0, 0)),
        out_shape=jax.ShapeDtypeStruct((B, 2), jnp.float32),
        scratch_shapes=[pltpu.VMEM((B, HID), jnp.float32),
                        pltpu.VMEM((B, HID), jnp.float32),
                        pltpu.VMEM((1, B), jnp.float32)],
    )(raw2, den2, bat2, gf, *consts)


def _attn_proj(a):
    heads = a.shape[0]
    eye = jnp.eye(heads, dtype=jnp.float32)
    m = (eye[:, None, :] * a[:, :, None]).reshape(heads * HID, heads)
    return jnp.pad(m, ((0, 0), (0, 16 - heads)))


_JNP_TAIL = False
_JNP_DENSE = False


def _jnp_dense_path(x, graph_features, p, src3, dst3, batch):
    def pad16(a):
        return jnp.pad(a, ((0, 0), (0, 16 - a.shape[1])))

    def gat(hin, W, a_s, a_d, heads):
        h = hin @ W
        hr = h.reshape(N, heads, HID)
        es = pad16((hr * a_s[None]).sum(-1))
        ed = pad16((hr * a_d[None]).sum(-1))
        extw, den = _sc_attention(es, ed, src3, dst3)
        harr = [h[:, i * 128:(i + 1) * 128] for i in range(max(1, heads * HID // 128))]
        if heads * HID < 128:
            harr = [h]
        raws = _sc_agg_chunks(harr, extw, src3, dst3, heads)
        raw = jnp.concatenate([o[0, :N] + o[1, :N] for o in raws], axis=1)
        denf = den[0, :N, :heads] + den[1, :N, :heads]
        out = raw.reshape(N, heads, HID) / denf[:, :, None]
        return out.reshape(N, heads * HID)

    z0 = jax.nn.relu((gat(x, p['W0'], p['as0'], p['ad0'], HEADS) + p['b0'])
                     * BNS * p['g0'] + p['be0'])
    z1 = jax.nn.relu((gat(z0, p['W1'], p['as1'], p['ad1'], HEADS) + p['b1'])
                     * BNS * p['g1'] + p['be1'])
    hin2 = z0 + z1
    h3 = gat(hin2, p['W2'], p['as2'], p['ad2'], 1) + p['b2']
    h = jax.nn.relu(h3 * BNS * p['g2'] + p['be2'])
    counts = jax.ops.segment_sum(jnp.ones((N, 1), jnp.float32), batch, num_segments=B)
    mean_pool = jax.ops.segment_sum(h, batch, num_segments=B) / jnp.maximum(counts, 1.0)
    max_pool = jax.ops.segment_max(h, batch, num_segments=B)
    gf = jax.nn.relu(graph_features @ p['Wg'] + p['bg'])
    c = jnp.concatenate([mean_pool, max_pool, gf], axis=1)
    z = jax.nn.relu((c @ p['Wr1'] + p['br1']) * BNS * p['gr1'] + p['ber1'])
    z = jax.nn.relu((z @ p['Wr2'] + p['br2']) * BNS * p['gr2'] + p['ber2'])
    return z @ p['Wr3'] + p['br3']


def kernel(x, graph_features, params, edge_index, batch):
    p = params
    loop = jnp.arange(N, dtype=edge_index.dtype)
    padv = jnp.arange(EP - E_TOT, dtype=edge_index.dtype) % N
    src_p = jnp.concatenate([edge_index[0], loop, padv]).reshape(NW, NBLK, KE)
    dst_p = jnp.concatenate([edge_index[1], loop, padv]).reshape(NW, NBLK, KE)
    x8 = jnp.pad(x, ((0, 0), (0, 5)))
    w0p = jnp.pad(p['W0'], ((0, 5), (0, 0)))
    bat2 = batch.reshape(N, 1)

    def vec(v):
        return jnp.reshape(v, (1, -1))

    if _JNP_DENSE:
        return _jnp_dense_path(x, graph_features, p, src_p, dst_p, batch)
    h0c0, h0c1, h0c2, h0c3, es0, ed0 = _tc0(x8, w0p, p['as0'], p['ad0'])
    ext0, den0 = _sc_attention(es0, ed0, src_p, dst_p)
    raw0 = _sc_agg_chunks([h0c0, h0c1, h0c2, h0c3], ext0, src_p, dst_p, HEADS)
    t1 = _tc1(raw0, den0, vec(p['b0']), vec(p['g0']), vec(p['be0']),
              p['W1'], p['as1'], p['ad1'])
    zres, h1c, (es1, ed1) = t1[:4], t1[4:8], t1[8:]
    ext1, den1 = _sc_attention(es1, ed1, src_p, dst_p)
    raw1 = _sc_agg_chunks(list(h1c), ext1, src_p, dst_p, HEADS)
    h2, es2, ed2 = _tc2(raw1, den1, list(zres), vec(p['b1']), vec(p['g1']),
                        vec(p['be1']), p['W2'], p['as2'], p['ad2'])
    ext2, den2 = _sc_attention(es2, ed2, src_p, dst_p)
    raw2 = _sc_agg_chunks([h2], ext2, src_p, dst_p, 1)
    if _JNP_TAIL:
        den = den2[0, :N, :1] + den2[1, :N, :1]
        h3 = (raw2[0][0, :N] + raw2[0][1, :N]) / den + p['b2']
        h = jax.nn.relu(h3 * BNS * p['g2'] + p['be2'])
        counts = jax.ops.segment_sum(jnp.ones((N, 1), jnp.float32), batch, num_segments=B)
        mean_pool = jax.ops.segment_sum(h, batch, num_segments=B) / jnp.maximum(counts, 1.0)
        max_pool = jax.ops.segment_max(h, batch, num_segments=B)
        gf = jax.nn.relu(graph_features @ p['Wg'] + p['bg'])
        c = jnp.concatenate([mean_pool, max_pool, gf], axis=1)
        z = jax.nn.relu((c @ p['Wr1'] + p['br1']) * BNS * p['gr1'] + p['ber1'])
        z = jax.nn.relu((z @ p['Wr2'] + p['br2']) * BNS * p['gr2'] + p['ber2'])
        return z @ p['Wr3'] + p['br3']
    return _tc3(raw2[0], den2, bat2, graph_features, p)
